# baseline (speedup 1.0000x reference)
import functools

import jax
import jax.numpy as jnp
from jax import lax
from jax.experimental import pallas as pl
from jax.experimental.pallas import tpu as pltpu
from jax.experimental.pallas import tpu_sc as plsc

_BATCH = 4096
_SEQ = 200
_D = 64
_NC, _NS = 2, 16
_NW = _NC * _NS
_BB = _BATCH // _NW
_L = 16


def _transpose_block(rows_v, trans_v, iota):
    @plsc.parallel_loop(0, _BB, unroll=16)
    def _b(b):
        b_vec = jnp.full((_L,), 0, jnp.int32) + b
        for e0 in range(0, _D, _L):
            x = rows_v[b, pl.ds(e0, _L)]
            plsc.store_scatter(trans_v, [e0 + iota, b_vec], x)


def _sc_body(p_t, h_t, table, p_out, h_out,
             idx_v, rows0, rows1, trans0, trans1,
             isem, gsem0, gsem1, osem0, osem1):
    wid = lax.axis_index("s") * _NC + lax.axis_index("c")
    rows = (rows0, rows1)
    trans = (trans0, trans1)
    gsems = (gsem0, gsem1)
    osems = (osem0, osem1)
    iota = lax.iota(jnp.int32, _L)

    for src, dst in ((p_t, p_out), (h_t, h_out)):
        pltpu.async_copy(src.at[:, pl.ds(wid * _BB, _BB)], idx_v, isem).wait()

        def fire_gather(s, b):
            return pltpu.async_copy(table.at[idx_v.at[s]], rows[b], gsems[b])

        def fire_write(s, b):
            for eh in range(8):
                pltpu.async_copy(
                    trans[b].at[pl.ds(eh * 8, 8), pl.ds(0, _BB)],
                    dst.at[s, eh, wid], osems[b])

        def drain_write(b):
            for eh in range(8):
                pltpu.make_async_copy(
                    trans[b].at[pl.ds(0, 8), pl.ds(0, _BB)],
                    dst.at[0, 0, wid], osems[b]).wait()

        def drain_gather(b):
            pltpu.make_async_copy(table.at[idx_v.at[0]], rows[b],
                                  gsems[b]).wait()

        fire_gather(0, 0)
        fire_gather(1, 1)

        @pl.loop(0, _SEQ, step=2)
        def _steady(s0):
            for b in range(2):
                s = s0 + b
                drain_gather(b)

                @pl.when(s0 >= 2)
                def _():
                    drain_write(b)

                _transpose_block(rows[b], trans[b], iota)

                @pl.when(s + 2 < _SEQ)
                def _():
                    fire_gather(s + 2, b)

                fire_write(s, b)

        for b in range(2):
            drain_write(b)


@functools.partial(
    pl.kernel,
    out_type=(jax.ShapeDtypeStruct((_SEQ, 8, _NW, 8, _BB), jnp.float32),
              jax.ShapeDtypeStruct((_SEQ, 8, _NW, 8, _BB), jnp.float32)),
    mesh=plsc.VectorSubcoreMesh(core_axis_name="c", subcore_axis_name="s"),
    compiler_params=pltpu.CompilerParams(use_tc_tiling_on_sc=False,
                                         needs_layout_passes=False),
    scratch_types=[
        pltpu.VMEM((_SEQ, _BB), jnp.int32),
        pltpu.VMEM((_BB, _D), jnp.float32),
        pltpu.VMEM((_BB, _D), jnp.float32),
        pltpu.VMEM((_D, _BB + 1), jnp.float32),
        pltpu.VMEM((_D, _BB + 1), jnp.float32),
        pltpu.SemaphoreType.DMA,
        pltpu.SemaphoreType.DMA,
        pltpu.SemaphoreType.DMA,
        pltpu.SemaphoreType.DMA,
        pltpu.SemaphoreType.DMA,
    ],
)
def _embed_lookup(p_t, h_t, table, p_out, h_out,
                  idx_v, rows0, rows1, trans0, trans1,
                  isem, gsem0, gsem1, osem0, osem1):
    _sc_body(p_t, h_t, table, p_out, h_out,
             idx_v, rows0, rows1, trans0, trans1,
             isem, gsem0, gsem1, osem0, osem1)


def kernel(premises, hypotheses, labels, table):
    p_raw, h_raw = _embed_lookup(premises.T, hypotheses.T, table)
    p_emb = p_raw.transpose(2, 4, 0, 1, 3).reshape(_BATCH, _SEQ, _D)
    h_emb = h_raw.transpose(2, 4, 0, 1, 3).reshape(_BATCH, _SEQ, _D)
    return (p_emb, h_emb, labels)


# speedup vs baseline: 1.0198x; 1.0198x over previous
import functools

import jax
import jax.numpy as jnp
from jax import lax
from jax.experimental import pallas as pl
from jax.experimental.pallas import tpu as pltpu
from jax.experimental.pallas import tpu_sc as plsc

_BATCH = 4096
_SEQ = 200
_D = 64
_NC, _NS = 2, 16
_NW = _NC * _NS
_BB = _BATCH // _NW
_L = 16


def _transpose_block(rows_v, trans_v, iota):
    @plsc.parallel_loop(0, _BB, unroll=8)
    def _b(b):
        b_vec = jnp.full((_L,), 0, jnp.int32) + b
        for e0 in range(0, _D, _L):
            x = rows_v[b, pl.ds(e0, _L)]
            plsc.store_scatter(trans_v, [e0 + iota, b_vec], x)


def _sc_body(p_t, h_t, table, p_out, h_out,
             idx_v, rows0, rows1, trans0, trans1,
             isem, gsem0, gsem1, osem0, osem1):
    wid = lax.axis_index("s") * _NC + lax.axis_index("c")
    rows = (rows0, rows1)
    trans = (trans0, trans1)
    gsems = (gsem0, gsem1)
    osems = (osem0, osem1)
    iota = lax.iota(jnp.int32, _L)

    for src, dst in ((p_t, p_out), (h_t, h_out)):
        pltpu.async_copy(src.at[:, pl.ds(wid * _BB, _BB)], idx_v, isem).wait()

        def fire_gather(s, b):
            return pltpu.async_copy(table.at[idx_v.at[s]], rows[b], gsems[b])

        def fire_write(s, b):
            for eh in range(8):
                pltpu.async_copy(
                    trans[b].at[pl.ds(eh * 8, 8), pl.ds(0, _BB)],
                    dst.at[s, eh, wid], osems[b])

        def drain_write(b):
            for eh in range(8):
                pltpu.make_async_copy(
                    trans[b].at[pl.ds(0, 8), pl.ds(0, _BB)],
                    dst.at[0, 0, wid], osems[b]).wait()

        def drain_gather(b):
            pltpu.make_async_copy(table.at[idx_v.at[0]], rows[b],
                                  gsems[b]).wait()

        fire_gather(0, 0)
        fire_gather(1, 1)

        @pl.loop(0, _SEQ, step=2)
        def _steady(s0):
            for b in range(2):
                s = s0 + b
                drain_gather(b)

                @pl.when(s0 >= 2)
                def _():
                    drain_write(b)

                _transpose_block(rows[b], trans[b], iota)

                @pl.when(s + 2 < _SEQ)
                def _():
                    fire_gather(s + 2, b)

                fire_write(s, b)

        for b in range(2):
            drain_write(b)


@functools.partial(
    pl.kernel,
    out_type=(jax.ShapeDtypeStruct((_SEQ, 8, _NW, 8, _BB), jnp.float32),
              jax.ShapeDtypeStruct((_SEQ, 8, _NW, 8, _BB), jnp.float32)),
    mesh=plsc.VectorSubcoreMesh(core_axis_name="c", subcore_axis_name="s"),
    compiler_params=pltpu.CompilerParams(use_tc_tiling_on_sc=False,
                                         needs_layout_passes=False),
    scratch_types=[
        pltpu.VMEM((_SEQ, _BB), jnp.int32),
        pltpu.VMEM((_BB, _D), jnp.float32),
        pltpu.VMEM((_BB, _D), jnp.float32),
        pltpu.VMEM((_D, _BB + 1), jnp.float32),
        pltpu.VMEM((_D, _BB + 1), jnp.float32),
        pltpu.SemaphoreType.DMA,
        pltpu.SemaphoreType.DMA,
        pltpu.SemaphoreType.DMA,
        pltpu.SemaphoreType.DMA,
        pltpu.SemaphoreType.DMA,
    ],
)
def _embed_lookup(p_t, h_t, table, p_out, h_out,
                  idx_v, rows0, rows1, trans0, trans1,
                  isem, gsem0, gsem1, osem0, osem1):
    _sc_body(p_t, h_t, table, p_out, h_out,
             idx_v, rows0, rows1, trans0, trans1,
             isem, gsem0, gsem1, osem0, osem1)


def kernel(premises, hypotheses, labels, table):
    p_raw, h_raw = _embed_lookup(premises.T, hypotheses.T, table)
    p_emb = p_raw.transpose(2, 4, 0, 1, 3).reshape(_BATCH, _SEQ, _D)
    h_emb = h_raw.transpose(2, 4, 0, 1, 3).reshape(_BATCH, _SEQ, _D)
    return (p_emb, h_emb, labels)


# speedup vs baseline: 1.1616x; 1.1391x over previous
import functools

import jax
import jax.numpy as jnp
from jax import lax
from jax.experimental import pallas as pl
from jax.experimental.pallas import tpu as pltpu
from jax.experimental.pallas import tpu_sc as plsc

_BATCH = 4096
_SEQ = 200
_D = 64
_NC, _NS = 2, 16
_NW = _NC * _NS
_BB = _BATCH // _NW
_L = 16
_NB = 4


def _transpose_block(rows_v, trans_v, idx_vecs):
    @plsc.parallel_loop(0, _BB, unroll=8)
    def _b(b):
        b_vec = jnp.full((_L,), 0, jnp.int32) + b
        for k, (ehi_vec, elo_vec) in enumerate(idx_vecs):
            x = rows_v[b, pl.ds(k * _L, _L)]
            plsc.store_scatter(trans_v, [ehi_vec, elo_vec, b_vec], x)


def _sc_body(p_t, h_t, table, p_out, h_out, idx_v, rows, trans, isem,
             gsems, osems):
    wid = lax.axis_index("s") * _NC + lax.axis_index("c")
    iota = lax.iota(jnp.int32, _L)
    idx_vecs = [((k * _L + iota) // 8, (k * _L + iota) % 8)
                for k in range(_D // _L)]

    for src, dst in ((p_t, p_out), (h_t, h_out)):
        pltpu.async_copy(src.at[:, pl.ds(wid * _BB, _BB)], idx_v, isem).wait()

        def fire_gather(s, b):
            return pltpu.async_copy(table.at[idx_v.at[s]], rows[b], gsems[b])

        def fire_write(s, b):
            pltpu.async_copy(trans[b].at[:, :, pl.ds(0, _BB)],
                             dst.at[s, :, wid], osems[b])

        def drain_write(b):
            pltpu.make_async_copy(trans[b].at[:, :, pl.ds(0, _BB)],
                                  dst.at[0, :, wid], osems[b]).wait()

        def drain_gather(b):
            pltpu.make_async_copy(table.at[idx_v.at[0]], rows[b],
                                  gsems[b]).wait()

        for s in range(_NB - 1):
            fire_gather(s, s)

        @pl.loop(0, _SEQ, step=_NB)
        def _steady(s0):
            for j in range(_NB):
                s = s0 + j
                b = j
                drain_gather(b)

                @pl.when(s0 >= _NB)
                def _():
                    drain_write(b)

                @pl.when(s + _NB - 1 < _SEQ)
                def _():
                    fire_gather(s + _NB - 1, (b + _NB - 1) % _NB)

                _transpose_block(rows[b], trans[b], idx_vecs)
                fire_write(s, b)

        for b in range(_NB):
            drain_write(b)


@functools.partial(
    pl.kernel,
    out_type=(jax.ShapeDtypeStruct((_SEQ, 8, _NW, 8, _BB), jnp.float32),
              jax.ShapeDtypeStruct((_SEQ, 8, _NW, 8, _BB), jnp.float32)),
    mesh=plsc.VectorSubcoreMesh(core_axis_name="c", subcore_axis_name="s"),
    compiler_params=pltpu.CompilerParams(use_tc_tiling_on_sc=False,
                                         needs_layout_passes=False),
    scratch_types=(
        [pltpu.VMEM((_SEQ, _BB), jnp.int32)]
        + [pltpu.VMEM((_BB, _D), jnp.float32)] * _NB
        + [pltpu.VMEM((8, 8, _BB + 1), jnp.float32)] * _NB
        + [pltpu.SemaphoreType.DMA] * (1 + 2 * _NB)
    ),
)
def _embed_lookup(p_t, h_t, table, p_out, h_out, idx_v, *bufs):
    rows = bufs[:_NB]
    trans = bufs[_NB:2 * _NB]
    isem = bufs[2 * _NB]
    gsems = bufs[2 * _NB + 1:2 * _NB + 1 + _NB]
    osems = bufs[2 * _NB + 1 + _NB:]
    _sc_body(p_t, h_t, table, p_out, h_out, idx_v, rows, trans, isem,
             gsems, osems)


def kernel(premises, hypotheses, labels, table):
    p_raw, h_raw = _embed_lookup(premises.T, hypotheses.T, table)
    p_emb = p_raw.transpose(2, 4, 0, 1, 3).reshape(_BATCH, _SEQ, _D)
    h_emb = h_raw.transpose(2, 4, 0, 1, 3).reshape(_BATCH, _SEQ, _D)
    return (p_emb, h_emb, labels)


# speedup vs baseline: 1.1695x; 1.0068x over previous
import functools

import jax
import jax.numpy as jnp
from jax import lax
from jax.experimental import pallas as pl
from jax.experimental.pallas import tpu as pltpu
from jax.experimental.pallas import tpu_sc as plsc

_BATCH = 4096
_SEQ = 200
_D = 64
_NC, _NS = 2, 16
_NW = _NC * _NS
_BB = _BATCH // _NW
_L = 16
_NB = 5


def _transpose_block(rows_v, trans_v, idx_vecs):
    @plsc.parallel_loop(0, _BB, unroll=8)
    def _b(b):
        b_vec = jnp.full((_L,), 0, jnp.int32) + b
        for k, (ehi_vec, elo_vec) in enumerate(idx_vecs):
            x = rows_v[b, pl.ds(k * _L, _L)]
            plsc.store_scatter(trans_v, [ehi_vec, elo_vec, b_vec], x)


def _sc_body(p_t, h_t, table, p_out, h_out, idx_v, rows, trans, isem,
             gsems, osems):
    wid = lax.axis_index("s") * _NC + lax.axis_index("c")
    iota = lax.iota(jnp.int32, _L)
    idx_vecs = [((k * _L + iota) // 8, (k * _L + iota) % 8)
                for k in range(_D // _L)]

    for src, dst in ((p_t, p_out), (h_t, h_out)):
        pltpu.async_copy(src.at[:, pl.ds(wid * _BB, _BB)], idx_v, isem).wait()

        def fire_gather(s, b):
            return pltpu.async_copy(table.at[idx_v.at[s]], rows[b], gsems[b])

        def fire_write(s, b):
            pltpu.async_copy(trans[b].at[:, :, pl.ds(0, _BB)],
                             dst.at[s, :, wid], osems[b])

        def drain_write(b):
            pltpu.make_async_copy(trans[b].at[:, :, pl.ds(0, _BB)],
                                  dst.at[0, :, wid], osems[b]).wait()

        def drain_gather(b):
            pltpu.make_async_copy(table.at[idx_v.at[0]], rows[b],
                                  gsems[b]).wait()

        for s in range(_NB - 1):
            fire_gather(s, s)

        @pl.loop(0, _SEQ, step=_NB)
        def _steady(s0):
            for j in range(_NB):
                s = s0 + j
                b = j
                drain_gather(b)

                @pl.when(s0 >= _NB)
                def _():
                    drain_write(b)

                @pl.when(s + _NB - 1 < _SEQ)
                def _():
                    fire_gather(s + _NB - 1, (b + _NB - 1) % _NB)

                _transpose_block(rows[b], trans[b], idx_vecs)
                fire_write(s, b)

        for b in range(_NB):
            drain_write(b)


@functools.partial(
    pl.kernel,
    out_type=(jax.ShapeDtypeStruct((_SEQ, 8, _NW, 8, _BB), jnp.float32),
              jax.ShapeDtypeStruct((_SEQ, 8, _NW, 8, _BB), jnp.float32)),
    mesh=plsc.VectorSubcoreMesh(core_axis_name="c", subcore_axis_name="s"),
    compiler_params=pltpu.CompilerParams(use_tc_tiling_on_sc=False,
                                         needs_layout_passes=False),
    scratch_types=(
        [pltpu.VMEM((_SEQ, _BB), jnp.int32)]
        + [pltpu.VMEM((_BB, _D), jnp.float32)] * _NB
        + [pltpu.VMEM((8, 8, _BB + 1), jnp.float32)] * _NB
        + [pltpu.SemaphoreType.DMA] * (1 + 2 * _NB)
    ),
)
def _embed_lookup(p_t, h_t, table, p_out, h_out, idx_v, *bufs):
    rows = bufs[:_NB]
    trans = bufs[_NB:2 * _NB]
    isem = bufs[2 * _NB]
    gsems = bufs[2 * _NB + 1:2 * _NB + 1 + _NB]
    osems = bufs[2 * _NB + 1 + _NB:]
    _sc_body(p_t, h_t, table, p_out, h_out, idx_v, rows, trans, isem,
             gsems, osems)


def kernel(premises, hypotheses, labels, table):
    p_raw, h_raw = _embed_lookup(premises.T, hypotheses.T, table)
    p_emb = p_raw.transpose(2, 4, 0, 1, 3).reshape(_BATCH, _SEQ, _D)
    h_emb = h_raw.transpose(2, 4, 0, 1, 3).reshape(_BATCH, _SEQ, _D)
    return (p_emb, h_emb, labels)
